# baseline (speedup 1.0000x reference)
import jax
import jax.numpy as jnp
from jax import lax
from jax.experimental import pallas as pl
from jax.experimental.pallas import tpu as pltpu
from jax.experimental.pallas import tpu_sc as plsc

N = 10000
E = 320000
D = 128
DH = D // 2

NC = 2
NS = 16
EDGES_PER_TILE = E // NS
CHUNK = 80
NCHUNK = EDGES_PER_TILE // CHUNK
ROW_STRIDE = 624
ROW_SIZE = 640


def _mm_body(x_ref, w_ref, b_ref, o_ref):
    o_ref[...] = (
        jnp.dot(x_ref[...], w_ref[...], preferred_element_type=jnp.float32)
        + b_ref[...]
    )


def _matmul(x, W, b):
    return pl.pallas_call(
        _mm_body,
        grid=(10,),
        in_specs=[
            pl.BlockSpec((1000, D), lambda i: (i, 0)),
            pl.BlockSpec((D, D), lambda i: (0, 0)),
            pl.BlockSpec((1, D), lambda i: (0, 0)),
        ],
        out_specs=pl.BlockSpec((1000, D), lambda i: (i, 0)),
        out_shape=jax.ShapeDtypeStruct((N, D), jnp.float32),
    )(x, W, b.reshape(1, D))


NBUF = 4
NIB = 8
LG = 2


def _sc_body(h_hbm, idx_hbm, w_hbm, zero_hbm, out_hbm,
             idxr, wr, rows, h_sh, acc_sh, si, sw, sg, ss):
    c = lax.axis_index("c")
    s = lax.axis_index("s")
    row0 = s * ROW_STRIDE
    pltpu.sync_copy(zero_hbm.at[pl.ds(row0, ROW_SIZE)],
                    acc_sh.at[pl.ds(row0, ROW_SIZE)])
    pltpu.sync_copy(h_hbm.at[c, pl.ds(row0, ROW_SIZE)],
                    h_sh.at[pl.ds(row0, ROW_SIZE)])
    plsc.subcore_barrier()

    def iissue(i, b):
        pltpu.async_copy(idx_hbm.at[s, i], idxr.at[b], si.at[b])
        pltpu.async_copy(w_hbm.at[s, i], wr.at[b], sw.at[b])

    def iwait(i, b):
        pltpu.make_async_copy(idx_hbm.at[s, i], idxr.at[b],
                              si.at[b]).wait()
        pltpu.make_async_copy(w_hbm.at[s, i], wr.at[b],
                              sw.at[b]).wait()

    def gsrc(par, ib):
        if par == 0:
            return h_hbm.at[c].at[idxr.at[ib, 0]]
        return h_sh.at[idxr.at[ib, 0]]

    def gissue(par, b, ib):
        pltpu.async_copy(gsrc(par, ib), rows.at[b], sg.at[b])

    def gwait(par, b, ib):
        pltpu.make_async_copy(gsrc(par, ib), rows.at[b], sg.at[b]).wait()

    def sissue(i, b, ib):
        pltpu.async_copy(rows.at[b], acc_sh.at[idxr.at[ib, 1]], ss.at[b],
                         add=True)

    def swait(i, b, ib):
        pltpu.make_async_copy(rows.at[b], acc_sh.at[idxr.at[ib, 1]],
                              ss.at[b]).wait()

    def scale(b, ib):
        for g in range(CHUNK // 16):
            w16 = wr[ib, pl.ds(g * 16, 16)]
            for j in range(16):
                wj = w16.at[jnp.full((16,), j, jnp.int32)].get(
                    mode="promise_in_bounds")
                e = g * 16 + j
                for blk in range(DH // 16):
                    r = rows[b, e, pl.ds(blk * 16, 16)]
                    rows[b, e, pl.ds(blk * 16, 16)] = r * wj

    for i in range(NIB):
        iissue(i, i)
    for i in range(LG):
        iwait(i, i)
        gissue(i % 2, i % NBUF, i)

    n_groups = (NCHUNK + NIB - 1) // NIB

    def main_body(ii, carry):
        ibase = ii * NIB
        for v in range(NIB):
            i = ibase + v

            @pl.when(i < NCHUNK)
            def _():
                gwait(v % 2, v % NBUF, v)

            @pl.when(jnp.logical_and(i < NCHUNK, i >= LG))
            def _():
                swait(i - LG, (v - LG) % NBUF, (v - LG) % NIB)

            @pl.when(jnp.logical_and(i >= LG, i + NIB - LG < NCHUNK))
            def _():
                iissue(i + NIB - LG, (v - LG) % NIB)

            @pl.when(jnp.logical_and(i < NCHUNK, i + LG < NCHUNK))
            def _():
                iwait(i + LG, (v + LG) % NIB)
                gissue((v + LG) % 2, (v + LG) % NBUF, (v + LG) % NIB)

            @pl.when(i < NCHUNK)
            def _():
                scale(v % NBUF, v)
                sissue(i, v % NBUF, v)

        return carry

    lax.fori_loop(0, n_groups, main_body, 0)

    for j in range(NCHUNK - LG, NCHUNK):
        swait(j, j % NBUF, j % NIB)

    plsc.subcore_barrier()
    pltpu.sync_copy(acc_sh.at[pl.ds(row0, ROW_SIZE)],
                    out_hbm.at[c, pl.ds(row0, ROW_SIZE)])


def _scatter(h3, src, dst, w, zeros):
    packed = (jnp.stack([src, dst], axis=0)
              .reshape(2, NS, NCHUNK, CHUNK)
              .transpose(1, 2, 0, 3))
    mesh = plsc.VectorSubcoreMesh(core_axis_name="c", subcore_axis_name="s")
    return pl.kernel(
        _sc_body,
        out_type=jax.ShapeDtypeStruct((NC, N, DH), jnp.float32),
        mesh=mesh,
        compiler_params=pltpu.CompilerParams(use_tc_tiling_on_sc=False),
        scratch_types=[
            pltpu.VMEM((NIB, 2, CHUNK), jnp.int32),
            pltpu.VMEM((NIB, CHUNK), jnp.float32),
            pltpu.VMEM((NBUF, CHUNK, DH), jnp.float32),
            pltpu.VMEM_SHARED((N, DH), jnp.float32),
            pltpu.VMEM_SHARED((N, DH), jnp.float32),
            pltpu.SemaphoreType.DMA((NIB,)),
            pltpu.SemaphoreType.DMA((NIB,)),
            pltpu.SemaphoreType.DMA((NBUF,)),
            pltpu.SemaphoreType.DMA((NBUF,)),
        ],
    )(h3, packed, w.reshape(NS, NCHUNK, CHUNK), zeros)


def kernel(input, edge_index, edge_weight, W, b):
    src = edge_index[0].astype(jnp.int32)
    dst = edge_index[1].astype(jnp.int32)
    h = _matmul(input, W, b)
    h3 = h.reshape(N, NC, DH).transpose(1, 0, 2)
    zeros = jnp.zeros((N, DH), jnp.float32)
    parts = _scatter(h3, src, dst, edge_weight, zeros)
    return jnp.concatenate([parts[0], parts[1]], axis=1)
